# baseline (speedup 1.0000x reference)
import functools

import jax
import jax.numpy as jnp
from jax import lax
from jax.experimental import pallas as pl
from jax.experimental.pallas import tpu as pltpu
from jax.experimental.pallas import tpu_sc as plsc

N = 10000
E = 160000
DF = 256
H = 256
C = 16
MIN_NORM = 1e-15
EPS = 4e-3

NC = 2
NS = 16
SLAB = 624
TAIL_OFF = NS * SLAB
TAIL = N - TAIL_OFF

EB = 128
EP = 163840
NP = 10016
NB1 = EP // NS // EB
NB2 = EP // (NC * NS) // EB
_HW = 128

_MESH = plsc.VectorSubcoreMesh(core_axis_name="c", subcore_axis_name="s")


def _zero_acc(zeros_hbm, acc_sh, s):
    pltpu.sync_copy(zeros_hbm.at[pl.ds(0, SLAB)],
                    acc_sh.at[pl.ds(s * SLAB, SLAB)])

    @pl.when(s == NS - 1)
    def _():
        pltpu.sync_copy(zeros_hbm.at[pl.ds(0, NP - TAIL_OFF)],
                        acc_sh.at[pl.ds(TAIL_OFF, NP - TAIL_OFF)])


def _write_out(acc_sh, out_hbm, c, s):
    pltpu.sync_copy(acc_sh.at[pl.ds(s * SLAB, SLAB)],
                    out_hbm.at[pl.ds(c * N + s * SLAB, SLAB)])

    @pl.when(s == NS - 1)
    def _():
        pltpu.sync_copy(acc_sh.at[pl.ds(TAIL_OFF, TAIL)],
                        out_hbm.at[pl.ds(c * N + TAIL_OFF, TAIL)])


@functools.partial(
    pl.kernel,
    out_type=jax.ShapeDtypeStruct((NC * N, C), jnp.float32),
    mesh=_MESH,
    scratch_types=[
        pltpu.VMEM((NB2, EB), jnp.int32),
        pltpu.VMEM((EB, C), jnp.float32),
        pltpu.VMEM_SHARED((NP, C), jnp.float32),
    ],
    compiler_params=pltpu.CompilerParams(use_tc_tiling_on_sc=False),
)
def _sc_deg(dst_hbm, ones_hbm, zeros_hbm, out_hbm, dst_v, ones_v, acc_sh):
    c = lax.axis_index("c")
    s = lax.axis_index("s")
    _zero_acc(zeros_hbm, acc_sh, s)
    pltpu.sync_copy(ones_hbm, ones_v)
    pltpu.sync_copy(dst_hbm.at[c * NS + s], dst_v)
    plsc.subcore_barrier()

    def body(j, carry):
        pltpu.sync_copy(ones_v, acc_sh.at[dst_v.at[j]], add=True)
        return carry

    lax.fori_loop(0, NB2, body, 0)
    plsc.subcore_barrier()
    _write_out(acc_sh, out_hbm, c, s)


@functools.partial(
    pl.kernel,
    out_type=jax.ShapeDtypeStruct((NC * N, _HW), jnp.float32),
    mesh=_MESH,
    scratch_types=[
        pltpu.VMEM((NB1 // 2, EB), jnp.int32),
        pltpu.VMEM((NB1 // 2, EB), jnp.int32),
        pltpu.VMEM((EB, _HW), jnp.float32),
        pltpu.VMEM((EB, _HW), jnp.float32),
        pltpu.VMEM_SHARED((NP, _HW), jnp.float32),
        pltpu.SemaphoreType.DMA,
        pltpu.SemaphoreType.DMA,
        pltpu.SemaphoreType.DMA,
        pltpu.SemaphoreType.DMA,
    ],
)
def _sc_agg1(g_hbm, src2_hbm, dst_hbm, out_hbm,
             src_v, dst_v, buf_a, buf_b, acc_sh, sem_a, sem_b,
             sem_sa, sem_sb):
    c = lax.axis_index("c")
    s = lax.axis_index("s")
    zv = jnp.zeros((16,), jnp.float32)

    def zrow(i, carry):
        for k in range(8):
            buf_a[i, pl.ds(k * 16, 16)] = zv
        return carry

    lax.fori_loop(0, EB, zrow, 0)
    for off, nrows in ((0, 128), (128, 128), (256, 128), (384, 128),
                       (512, SLAB - 512)):
        pltpu.sync_copy(buf_a.at[pl.ds(0, nrows)],
                        acc_sh.at[pl.ds(s * SLAB + off, nrows)])

    @pl.when(s == NS - 1)
    def _():
        pltpu.sync_copy(buf_a.at[pl.ds(0, NP - TAIL_OFF)],
                        acc_sh.at[pl.ds(TAIL_OFF, NP - TAIL_OFF)])

    plsc.subcore_barrier()

    def body(i, carry):
        ja = 2 * i
        jb = 2 * i + 1
        da = pltpu.async_copy(g_hbm.at[src_v.at[ja]], buf_a, sem_a)
        db = pltpu.async_copy(g_hbm.at[src_v.at[jb]], buf_b, sem_b)
        da.wait()
        pltpu.sync_copy(buf_a, acc_sh.at[dst_v.at[ja]], add=True)
        db.wait()
        pltpu.sync_copy(buf_b, acc_sh.at[dst_v.at[jb]], add=True)
        return carry

    for hf in range(2):
        pltpu.sync_copy(src2_hbm.at[c, s, pl.ds(hf * (NB1 // 2), NB1 // 2)],
                        src_v)
        pltpu.sync_copy(dst_hbm.at[s, pl.ds(hf * (NB1 // 2), NB1 // 2)],
                        dst_v)
        lax.fori_loop(0, NB1 // 4, body, 0)
    plsc.subcore_barrier()
    _write_out(acc_sh, out_hbm, c, s)


@functools.partial(
    pl.kernel,
    out_type=jax.ShapeDtypeStruct((NC * N, C), jnp.float32),
    mesh=_MESH,
    scratch_types=[
        pltpu.VMEM((NB2, EB), jnp.int32),
        pltpu.VMEM((NB2, EB), jnp.int32),
        pltpu.VMEM((EB, C), jnp.float32),
        pltpu.VMEM((EB, C), jnp.float32),
        pltpu.VMEM_SHARED((NP, C), jnp.float32),
        pltpu.SemaphoreType.DMA,
        pltpu.SemaphoreType.DMA,
    ],
    compiler_params=pltpu.CompilerParams(use_tc_tiling_on_sc=False),
)
def _sc_agg2(g2_hbm, src_hbm, dst_hbm, zeros_hbm, out_hbm,
             src_v, dst_v, buf_a, buf_b, acc_sh, sem_a, sem_b):
    c = lax.axis_index("c")
    s = lax.axis_index("s")
    _zero_acc(zeros_hbm, acc_sh, s)
    w = c * NS + s
    pltpu.sync_copy(src_hbm.at[w], src_v)
    pltpu.sync_copy(dst_hbm.at[w], dst_v)
    plsc.subcore_barrier()

    def body(i, carry):
        ja = 2 * i
        jb = 2 * i + 1
        da = pltpu.async_copy(g2_hbm.at[src_v.at[ja]], buf_a, sem_a)
        db = pltpu.async_copy(g2_hbm.at[src_v.at[jb]], buf_b, sem_b)
        da.wait()
        pltpu.sync_copy(buf_a, acc_sh.at[dst_v.at[ja]], add=True)
        db.wait()
        pltpu.sync_copy(buf_b, acc_sh.at[dst_v.at[jb]], add=True)
        return carry

    lax.fori_loop(0, NB2 // 2, body, 0)
    plsc.subcore_barrier()
    _write_out(acc_sh, out_hbm, c, s)


_BR = 1000
_GRID = N // _BR


def _dinv(dA_ref, dB_ref):
    d = 1.0 + dA_ref[:, :1] + dB_ref[:, :1]
    return lax.rsqrt(d)


def _scale_body(x_ref, w1_ref, dA_ref, dB_ref, g_ref):
    u = jnp.dot(x_ref[...], w1_ref[...], preferred_element_type=jnp.float32)
    g = u * _dinv(dA_ref, dB_ref)
    g_ref[0] = g[:, :_HW]
    g_ref[1] = g[:, _HW:]


def _l1_body(aggA, aggB, gA, gB, dA, dB, b1_ref, w2_ref, g2_ref):
    dinv = _dinv(dA, dB)
    agg = jnp.concatenate([aggA[...], aggB[...]], axis=1)
    g = jnp.concatenate([gA[...], gB[...]], axis=1)
    h1 = jax.nn.relu((agg + g) * dinv + b1_ref[...])
    v = jnp.dot(h1, w2_ref[...], preferred_element_type=jnp.float32)
    g2_ref[...] = v * dinv


def _fin_body(a2A, a2B, g2, dA, dB, b2_ref, wl_ref, bl_ref, o_ref):
    dinv = _dinv(dA, dB)
    h2 = (a2A[...] + a2B[...] + g2[...]) * dinv + b2_ref[...]
    flat = jnp.concatenate([h2 * h2[:, j:j + 1] for j in range(C)], axis=1)
    maxnorm = 1.0 - EPS
    norm = jnp.maximum(jnp.sqrt(jnp.sum(flat * flat, axis=1, keepdims=True)),
                       MIN_NORM)
    hy = jnp.where(norm > maxnorm, flat / norm * maxnorm, flat)
    pn = jnp.maximum(jnp.sqrt(jnp.sum(hy * hy, axis=1, keepdims=True)),
                     MIN_NORM)
    pc = jnp.clip(pn, -1.0 + 1e-7, 1.0 - 1e-7)
    atanh = 0.5 * jnp.log((1.0 + pc) / (1.0 - pc))
    he = (atanh / pn) * hy
    z = jnp.dot(he, wl_ref[...], preferred_element_type=jnp.float32) \
        + bl_ref[...]
    un = jnp.maximum(jnp.sqrt(jnp.sum(z * z, axis=1, keepdims=True)),
                     MIN_NORM)
    e = jnp.tanh(un) * z / un
    n2 = jnp.maximum(jnp.sqrt(jnp.sum(e * e, axis=1, keepdims=True)),
                     MIN_NORM)
    hh = jnp.where(n2 > maxnorm, e / n2 * maxnorm, e)
    m = jnp.max(hh, axis=1, keepdims=True)
    sh = hh - m
    lse = jnp.log(jnp.sum(jnp.exp(sh), axis=1, keepdims=True))
    o_ref[...] = sh - lse


def _row_spec(w):
    return pl.BlockSpec((_BR, w), lambda i: (i, 0))


def _rowB_spec(w):
    return pl.BlockSpec((_BR, w), lambda i: (i + _GRID, 0))


def _full_spec(a, b):
    return pl.BlockSpec((a, b), lambda i: (0, 0))


def kernel(x, edge_index, W1, b1, W2, b2, Wl, bl):
    src = edge_index[0].astype(jnp.int32)
    dst = edge_index[1].astype(jnp.int32)
    pad = EP - E
    src_p = jnp.concatenate([src, jnp.zeros((pad,), jnp.int32)])
    dst_p = jnp.concatenate([dst, jnp.full((pad,), N, jnp.int32)])
    src2_r = jnp.concatenate([src_p, src_p + N]).reshape(NC, NS, NB1, EB)
    dst1_r = dst_p.reshape(NS, NB1, EB)
    src2w_r = src_p.reshape(NC * NS, NB2, EB)
    dst2w_r = dst_p.reshape(NC * NS, NB2, EB)

    ones_d = jnp.ones((EB, C), jnp.float32)
    zeros_c = jnp.zeros((SLAB, C), jnp.float32)

    deg2 = _sc_deg(dst2w_r, ones_d, zeros_c)

    g_st = pl.pallas_call(
        _scale_body, grid=(_GRID,),
        in_specs=[_row_spec(DF), _full_spec(DF, H), _row_spec(C),
                  _rowB_spec(C)],
        out_specs=pl.BlockSpec((NC, _BR, _HW), lambda i: (0, i, 0)),
        out_shape=jax.ShapeDtypeStruct((NC, N, _HW), jnp.float32),
    )(x, W1, deg2, deg2)
    g_st = g_st.reshape(NC * N, _HW)

    agg1 = _sc_agg1(g_st, src2_r, dst1_r)

    g2 = pl.pallas_call(
        _l1_body, grid=(_GRID,),
        in_specs=[_row_spec(_HW), _rowB_spec(_HW), _row_spec(_HW),
                  _rowB_spec(_HW), _row_spec(C), _rowB_spec(C),
                  _full_spec(1, H), _full_spec(H, C)],
        out_specs=_row_spec(C),
        out_shape=jax.ShapeDtypeStruct((N, C), jnp.float32),
    )(agg1, agg1, g_st, g_st, deg2, deg2, b1.reshape(1, H), W2)

    agg2 = _sc_agg2(g2, src2w_r, dst2w_r, zeros_c)

    out = pl.pallas_call(
        _fin_body, grid=(_GRID,),
        in_specs=[_row_spec(C), _rowB_spec(C), _row_spec(C), _row_spec(C),
                  _rowB_spec(C), _full_spec(1, C), _full_spec(C * C, C),
                  _full_spec(1, C)],
        out_specs=_row_spec(C),
        out_shape=jax.ShapeDtypeStruct((N, C), jnp.float32),
    )(agg2, agg2, g2, deg2, deg2, b2.reshape(1, C), Wl, bl.reshape(1, C))
    return out


# speedup vs baseline: 1.0704x; 1.0704x over previous
import functools

import jax
import jax.numpy as jnp
from jax import lax
from jax.experimental import pallas as pl
from jax.experimental.pallas import tpu as pltpu
from jax.experimental.pallas import tpu_sc as plsc

N = 10000
E = 160000
DF = 256
H = 256
C = 16
MIN_NORM = 1e-15
EPS = 4e-3

NC = 2
NS = 16
SLAB = 624
TAIL_OFF = NS * SLAB
TAIL = N - TAIL_OFF

EB = 128
EP = 163840
NP = 10016
NB1 = EP // NS // EB
NB2 = EP // (NC * NS) // EB
_HW = 128

_MESH = plsc.VectorSubcoreMesh(core_axis_name="c", subcore_axis_name="s")


def _zero_acc(zeros_hbm, acc_sh, s):
    pltpu.sync_copy(zeros_hbm.at[pl.ds(0, SLAB)],
                    acc_sh.at[pl.ds(s * SLAB, SLAB)])

    @pl.when(s == NS - 1)
    def _():
        pltpu.sync_copy(zeros_hbm.at[pl.ds(0, NP - TAIL_OFF)],
                        acc_sh.at[pl.ds(TAIL_OFF, NP - TAIL_OFF)])


def _write_out(acc_sh, out_hbm, c, s):
    pltpu.sync_copy(acc_sh.at[pl.ds(s * SLAB, SLAB)],
                    out_hbm.at[pl.ds(c * N + s * SLAB, SLAB)])

    @pl.when(s == NS - 1)
    def _():
        pltpu.sync_copy(acc_sh.at[pl.ds(TAIL_OFF, TAIL)],
                        out_hbm.at[pl.ds(c * N + TAIL_OFF, TAIL)])


@functools.partial(
    pl.kernel,
    out_type=jax.ShapeDtypeStruct((NC * N, C), jnp.float32),
    mesh=_MESH,
    scratch_types=[
        pltpu.VMEM((NB2, EB), jnp.int32),
        pltpu.VMEM((EB, C), jnp.float32),
        pltpu.VMEM_SHARED((NP, C), jnp.float32),
    ],
    compiler_params=pltpu.CompilerParams(use_tc_tiling_on_sc=False),
)
def _sc_deg(dst_hbm, ones_hbm, zeros_hbm, out_hbm, dst_v, ones_v, acc_sh):
    c = lax.axis_index("c")
    s = lax.axis_index("s")
    _zero_acc(zeros_hbm, acc_sh, s)
    pltpu.sync_copy(ones_hbm, ones_v)
    pltpu.sync_copy(dst_hbm.at[c * NS + s], dst_v)
    plsc.subcore_barrier()

    def body(j, carry):
        pltpu.sync_copy(ones_v, acc_sh.at[dst_v.at[j]], add=True)
        return carry

    lax.fori_loop(0, NB2, body, 0)
    plsc.subcore_barrier()
    _write_out(acc_sh, out_hbm, c, s)


@functools.partial(
    pl.kernel,
    out_type=jax.ShapeDtypeStruct((NC * N, _HW), jnp.float32),
    mesh=_MESH,
    scratch_types=[
        pltpu.VMEM((NB1 // 2, EB), jnp.int32),
        pltpu.VMEM((NB1 // 2, EB), jnp.int32),
        pltpu.VMEM((EB, _HW), jnp.float32),
        pltpu.VMEM((EB, _HW), jnp.float32),
        pltpu.VMEM_SHARED((NP, _HW), jnp.float32),
        pltpu.SemaphoreType.DMA,
        pltpu.SemaphoreType.DMA,
        pltpu.SemaphoreType.DMA,
        pltpu.SemaphoreType.DMA,
    ],
)
def _sc_agg1(g_hbm, src2_hbm, dst_hbm, out_hbm,
             src_v, dst_v, buf_a, buf_b, acc_sh, sem_a, sem_b,
             sem_sa, sem_sb):
    c = lax.axis_index("c")
    s = lax.axis_index("s")
    zv = jnp.zeros((16,), jnp.float32)

    def zrow(i, carry):
        for k in range(8):
            buf_a[i, pl.ds(k * 16, 16)] = zv
        return carry

    lax.fori_loop(0, EB, zrow, 0)
    for off, nrows in ((0, 128), (128, 128), (256, 128), (384, 128),
                       (512, SLAB - 512)):
        pltpu.sync_copy(buf_a.at[pl.ds(0, nrows)],
                        acc_sh.at[pl.ds(s * SLAB + off, nrows)])

    @pl.when(s == NS - 1)
    def _():
        pltpu.sync_copy(buf_a.at[pl.ds(0, NP - TAIL_OFF)],
                        acc_sh.at[pl.ds(TAIL_OFF, NP - TAIL_OFF)])

    plsc.subcore_barrier()

    def body(i, carry):
        ja = 2 * i
        jb = 2 * i + 1
        da = pltpu.async_copy(g_hbm.at[src_v.at[ja]], buf_a, sem_a)
        db = pltpu.async_copy(g_hbm.at[src_v.at[jb]], buf_b, sem_b)
        da.wait()
        pltpu.sync_copy(buf_a, acc_sh.at[dst_v.at[ja]], add=True)
        db.wait()
        pltpu.sync_copy(buf_b, acc_sh.at[dst_v.at[jb]], add=True)
        return carry

    for hf in range(2):
        pltpu.sync_copy(src2_hbm.at[c, s, pl.ds(hf * (NB1 // 2), NB1 // 2)],
                        src_v)
        pltpu.sync_copy(dst_hbm.at[s, pl.ds(hf * (NB1 // 2), NB1 // 2)],
                        dst_v)
        lax.fori_loop(0, NB1 // 4, body, 0)
    plsc.subcore_barrier()
    _write_out(acc_sh, out_hbm, c, s)


@functools.partial(
    pl.kernel,
    out_type=jax.ShapeDtypeStruct((NC * N, C), jnp.float32),
    mesh=_MESH,
    scratch_types=[
        pltpu.VMEM((NB2, EB), jnp.int32),
        pltpu.VMEM((NB2, EB), jnp.int32),
        pltpu.VMEM((EB, C), jnp.float32),
        pltpu.VMEM((EB, C), jnp.float32),
        pltpu.VMEM_SHARED((NP, C), jnp.float32),
        pltpu.SemaphoreType.DMA,
        pltpu.SemaphoreType.DMA,
    ],
    compiler_params=pltpu.CompilerParams(use_tc_tiling_on_sc=False),
)
def _sc_agg2(g2_hbm, src_hbm, dst_hbm, zeros_hbm, out_hbm,
             src_v, dst_v, buf_a, buf_b, acc_sh, sem_a, sem_b):
    c = lax.axis_index("c")
    s = lax.axis_index("s")
    _zero_acc(zeros_hbm, acc_sh, s)
    w = c * NS + s
    pltpu.sync_copy(src_hbm.at[w], src_v)
    pltpu.sync_copy(dst_hbm.at[w], dst_v)
    plsc.subcore_barrier()

    def body(i, carry):
        ja = 2 * i
        jb = 2 * i + 1
        da = pltpu.async_copy(g2_hbm.at[src_v.at[ja]], buf_a, sem_a)
        db = pltpu.async_copy(g2_hbm.at[src_v.at[jb]], buf_b, sem_b)
        da.wait()
        pltpu.sync_copy(buf_a, acc_sh.at[dst_v.at[ja]], add=True)
        db.wait()
        pltpu.sync_copy(buf_b, acc_sh.at[dst_v.at[jb]], add=True)
        return carry

    lax.fori_loop(0, NB2 // 2, body, 0)
    plsc.subcore_barrier()
    _write_out(acc_sh, out_hbm, c, s)


_BR = 1000
_GRID = N // _BR


def _mm_body(x_ref, w_ref, o_ref):
    o_ref[...] = jnp.dot(x_ref[...], w_ref[...],
                         preferred_element_type=jnp.float32)


def _dinv(dA_ref, dB_ref):
    d = 1.0 + dA_ref[:, :1] + dB_ref[:, :1]
    return lax.rsqrt(d)


def _scale_body(u_ref, dA_ref, dB_ref, g_ref):
    g = u_ref[...] * _dinv(dA_ref, dB_ref)
    g_ref[0] = g[:, :_HW]
    g_ref[1] = g[:, _HW:]


def _l1_body(aggA, aggB, gA, gB, dA, dB, b1_ref, w2_ref, g2_ref):
    dinv = _dinv(dA, dB)
    agg = jnp.concatenate([aggA[...], aggB[...]], axis=1)
    g = jnp.concatenate([gA[...], gB[...]], axis=1)
    h1 = jax.nn.relu((agg + g) * dinv + b1_ref[...])
    v = jnp.dot(h1, w2_ref[...], preferred_element_type=jnp.float32)
    g2_ref[...] = v * dinv


def _fin_body(a2A, a2B, g2, dA, dB, b2_ref, wl_ref, bl_ref, o_ref):
    dinv = _dinv(dA, dB)
    h2 = (a2A[...] + a2B[...] + g2[...]) * dinv + b2_ref[...]
    flat = jnp.concatenate([h2 * h2[:, j:j + 1] for j in range(C)], axis=1)
    maxnorm = 1.0 - EPS
    norm = jnp.maximum(jnp.sqrt(jnp.sum(flat * flat, axis=1, keepdims=True)),
                       MIN_NORM)
    hy = jnp.where(norm > maxnorm, flat / norm * maxnorm, flat)
    pn = jnp.maximum(jnp.sqrt(jnp.sum(hy * hy, axis=1, keepdims=True)),
                     MIN_NORM)
    pc = jnp.clip(pn, -1.0 + 1e-7, 1.0 - 1e-7)
    atanh = 0.5 * jnp.log((1.0 + pc) / (1.0 - pc))
    he = (atanh / pn) * hy
    z = jnp.dot(he, wl_ref[...], preferred_element_type=jnp.float32) \
        + bl_ref[...]
    un = jnp.maximum(jnp.sqrt(jnp.sum(z * z, axis=1, keepdims=True)),
                     MIN_NORM)
    e = jnp.tanh(un) * z / un
    n2 = jnp.maximum(jnp.sqrt(jnp.sum(e * e, axis=1, keepdims=True)),
                     MIN_NORM)
    hh = jnp.where(n2 > maxnorm, e / n2 * maxnorm, e)
    m = jnp.max(hh, axis=1, keepdims=True)
    sh = hh - m
    lse = jnp.log(jnp.sum(jnp.exp(sh), axis=1, keepdims=True))
    o_ref[...] = sh - lse


def _row_spec(w):
    return pl.BlockSpec((_BR, w), lambda i: (i, 0))


def _rowB_spec(w):
    return pl.BlockSpec((_BR, w), lambda i: (i + _GRID, 0))


def _full_spec(a, b):
    return pl.BlockSpec((a, b), lambda i: (0, 0))


def kernel(x, edge_index, W1, b1, W2, b2, Wl, bl):
    src = edge_index[0].astype(jnp.int32)
    dst = edge_index[1].astype(jnp.int32)
    pad = EP - E
    src_p = jnp.concatenate([src, jnp.zeros((pad,), jnp.int32)])
    dst_p = jnp.concatenate([dst, jnp.full((pad,), N, jnp.int32)])
    src2_r = jnp.concatenate([src_p, src_p + N]).reshape(NC, NS, NB1, EB)
    dst1_r = dst_p.reshape(NS, NB1, EB)
    src2w_r = src_p.reshape(NC * NS, NB2, EB)
    dst2w_r = dst_p.reshape(NC * NS, NB2, EB)

    ones_d = jnp.ones((EB, C), jnp.float32)
    zeros_c = jnp.zeros((SLAB, C), jnp.float32)

    deg2 = _sc_deg(dst2w_r, ones_d, zeros_c)

    u = pl.pallas_call(
        _mm_body, grid=(_GRID,),
        in_specs=[_row_spec(DF), _full_spec(DF, H)],
        out_specs=_row_spec(H),
        out_shape=jax.ShapeDtypeStruct((N, H), jnp.float32),
    )(x, W1)

    g_st = pl.pallas_call(
        _scale_body, grid=(_GRID,),
        in_specs=[_row_spec(H), _row_spec(C), _rowB_spec(C)],
        out_specs=pl.BlockSpec((NC, _BR, _HW), lambda i: (0, i, 0)),
        out_shape=jax.ShapeDtypeStruct((NC, N, _HW), jnp.float32),
    )(u, deg2, deg2)
    g_st = g_st.reshape(NC * N, _HW)

    agg1 = _sc_agg1(g_st, src2_r, dst1_r)

    g2 = pl.pallas_call(
        _l1_body, grid=(_GRID,),
        in_specs=[_row_spec(_HW), _rowB_spec(_HW), _row_spec(_HW),
                  _rowB_spec(_HW), _row_spec(C), _rowB_spec(C),
                  _full_spec(1, H), _full_spec(H, C)],
        out_specs=_row_spec(C),
        out_shape=jax.ShapeDtypeStruct((N, C), jnp.float32),
    )(agg1, agg1, g_st, g_st, deg2, deg2, b1.reshape(1, H), W2)

    agg2 = _sc_agg2(g2, src2w_r, dst2w_r, zeros_c)

    out = pl.pallas_call(
        _fin_body, grid=(_GRID,),
        in_specs=[_row_spec(C), _rowB_spec(C), _row_spec(C), _row_spec(C),
                  _rowB_spec(C), _full_spec(1, C), _full_spec(C * C, C),
                  _full_spec(1, C)],
        out_specs=_row_spec(C),
        out_shape=jax.ShapeDtypeStruct((N, C), jnp.float32),
    )(agg2, agg2, g2, deg2, deg2, b2.reshape(1, C), Wl, bl.reshape(1, C))
    return out


# speedup vs baseline: 1.1525x; 1.0767x over previous
import functools

import jax
import jax.numpy as jnp
from jax import lax
from jax.experimental import pallas as pl
from jax.experimental.pallas import tpu as pltpu
from jax.experimental.pallas import tpu_sc as plsc

N = 10000
E = 160000
DF = 256
H = 256
C = 16
MIN_NORM = 1e-15
EPS = 4e-3

NC = 2
NS = 16
SLAB = 624
TAIL_OFF = NS * SLAB
TAIL = N - TAIL_OFF

EB = 128
EB1 = 64
EP = 163840
NP = 10016
NB1 = EP // NS // EB1
NBH = NB1 // 4
NB2 = EP // (NC * NS) // EB
_HW = 128

_MESH = plsc.VectorSubcoreMesh(core_axis_name="c", subcore_axis_name="s")


def _zero_acc(zeros_hbm, acc_sh, s):
    pltpu.sync_copy(zeros_hbm.at[pl.ds(0, SLAB)],
                    acc_sh.at[pl.ds(s * SLAB, SLAB)])

    @pl.when(s == NS - 1)
    def _():
        pltpu.sync_copy(zeros_hbm.at[pl.ds(0, NP - TAIL_OFF)],
                        acc_sh.at[pl.ds(TAIL_OFF, NP - TAIL_OFF)])


def _write_out(acc_sh, out_hbm, c, s):
    pltpu.sync_copy(acc_sh.at[pl.ds(s * SLAB, SLAB)],
                    out_hbm.at[pl.ds(c * N + s * SLAB, SLAB)])

    @pl.when(s == NS - 1)
    def _():
        pltpu.sync_copy(acc_sh.at[pl.ds(TAIL_OFF, TAIL)],
                        out_hbm.at[pl.ds(c * N + TAIL_OFF, TAIL)])


@functools.partial(
    pl.kernel,
    out_type=jax.ShapeDtypeStruct((NC * N, C), jnp.float32),
    mesh=_MESH,
    scratch_types=[
        pltpu.VMEM((NB2, EB), jnp.int32),
        pltpu.VMEM((EB, C), jnp.float32),
        pltpu.VMEM_SHARED((NP, C), jnp.float32),
    ],
    compiler_params=pltpu.CompilerParams(use_tc_tiling_on_sc=False),
)
def _sc_deg(dst_hbm, ones_hbm, zeros_hbm, out_hbm, dst_v, ones_v, acc_sh):
    c = lax.axis_index("c")
    s = lax.axis_index("s")
    _zero_acc(zeros_hbm, acc_sh, s)
    pltpu.sync_copy(ones_hbm, ones_v)
    pltpu.sync_copy(dst_hbm.at[c * NS + s], dst_v)
    plsc.subcore_barrier()

    def body(j, carry):
        pltpu.sync_copy(ones_v, acc_sh.at[dst_v.at[j]], add=True)
        return carry

    lax.fori_loop(0, NB2, body, 0)
    plsc.subcore_barrier()
    _write_out(acc_sh, out_hbm, c, s)


@functools.partial(
    pl.kernel,
    out_type=jax.ShapeDtypeStruct((NC * N, _HW), jnp.float32),
    mesh=_MESH,
    scratch_types=[
        pltpu.VMEM((NBH, EB1), jnp.int32),
        pltpu.VMEM((NBH, EB1), jnp.int32),
        pltpu.VMEM((EB1, _HW), jnp.float32),
        pltpu.VMEM((EB1, _HW), jnp.float32),
        pltpu.VMEM((EB1, _HW), jnp.float32),
        pltpu.VMEM((EB1, _HW), jnp.float32),
        pltpu.VMEM_SHARED((NP, _HW), jnp.float32),
        pltpu.SemaphoreType.DMA,
        pltpu.SemaphoreType.DMA,
        pltpu.SemaphoreType.DMA,
        pltpu.SemaphoreType.DMA,
    ],
)
def _sc_agg1(g_hbm, src2_hbm, dst_hbm, out_hbm,
             src_v, dst_v, b0, b1, b2, b3, acc_sh, s0, s1, s2, s3):
    c = lax.axis_index("c")
    s = lax.axis_index("s")
    zv = jnp.zeros((16,), jnp.float32)

    def zrow(i, carry):
        for k in range(8):
            b0[i, pl.ds(k * 16, 16)] = zv
        return carry

    lax.fori_loop(0, EB1, zrow, 0)
    for off in range(0, 576, EB1):
        pltpu.sync_copy(b0, acc_sh.at[pl.ds(s * SLAB + off, EB1)])
    pltpu.sync_copy(b0.at[pl.ds(0, SLAB - 576)],
                    acc_sh.at[pl.ds(s * SLAB + 576, SLAB - 576)])

    @pl.when(s == NS - 1)
    def _():
        pltpu.sync_copy(b0.at[pl.ds(0, NP - TAIL_OFF)],
                        acc_sh.at[pl.ds(TAIL_OFF, NP - TAIL_OFF)])

    plsc.subcore_barrier()

    bufs = (b0, b1, b2, b3)
    sems = (s0, s1, s2, s3)

    def body(i, carry):
        for k in range(4):
            j = 4 * i + k
            buf, sem = bufs[k], sems[k]
            pltpu.make_async_copy(g_hbm.at[src_v.at[j]], buf, sem).wait()
            pltpu.sync_copy(buf, acc_sh.at[dst_v.at[j]], add=True)
            nj = j + 4

            @pl.when(nj < NBH)
            def _():
                pltpu.async_copy(g_hbm.at[src_v.at[nj]], buf, sem)

        return carry

    for hf in range(4):
        pltpu.sync_copy(src2_hbm.at[c, s, pl.ds(hf * NBH, NBH)], src_v)
        pltpu.sync_copy(dst_hbm.at[s, pl.ds(hf * NBH, NBH)], dst_v)
        for k in range(4):
            pltpu.async_copy(g_hbm.at[src_v.at[k]], bufs[k], sems[k])
        lax.fori_loop(0, NBH // 4, body, 0)
    plsc.subcore_barrier()
    _write_out(acc_sh, out_hbm, c, s)


@functools.partial(
    pl.kernel,
    out_type=jax.ShapeDtypeStruct((NC * N, C), jnp.float32),
    mesh=_MESH,
    scratch_types=[
        pltpu.VMEM((NB2, EB), jnp.int32),
        pltpu.VMEM((NB2, EB), jnp.int32),
        pltpu.VMEM((EB, C), jnp.float32),
        pltpu.VMEM((EB, C), jnp.float32),
        pltpu.VMEM_SHARED((NP, C), jnp.float32),
        pltpu.SemaphoreType.DMA,
        pltpu.SemaphoreType.DMA,
    ],
    compiler_params=pltpu.CompilerParams(use_tc_tiling_on_sc=False),
)
def _sc_agg2(g2_hbm, src_hbm, dst_hbm, zeros_hbm, out_hbm,
             src_v, dst_v, buf_a, buf_b, acc_sh, sem_a, sem_b):
    c = lax.axis_index("c")
    s = lax.axis_index("s")
    _zero_acc(zeros_hbm, acc_sh, s)
    w = c * NS + s
    pltpu.sync_copy(src_hbm.at[w], src_v)
    pltpu.sync_copy(dst_hbm.at[w], dst_v)
    plsc.subcore_barrier()

    def body(i, carry):
        ja = 2 * i
        jb = 2 * i + 1
        da = pltpu.async_copy(g2_hbm.at[src_v.at[ja]], buf_a, sem_a)
        db = pltpu.async_copy(g2_hbm.at[src_v.at[jb]], buf_b, sem_b)
        da.wait()
        pltpu.sync_copy(buf_a, acc_sh.at[dst_v.at[ja]], add=True)
        db.wait()
        pltpu.sync_copy(buf_b, acc_sh.at[dst_v.at[jb]], add=True)
        return carry

    lax.fori_loop(0, NB2 // 2, body, 0)
    plsc.subcore_barrier()
    _write_out(acc_sh, out_hbm, c, s)


_BR = 1000
_GRID = N // _BR


def _mm_body(x_ref, w_ref, o_ref):
    o_ref[...] = jnp.dot(x_ref[...], w_ref[...],
                         preferred_element_type=jnp.float32)


def _dinv(dA_ref, dB_ref):
    d = 1.0 + dA_ref[:, :1] + dB_ref[:, :1]
    return lax.rsqrt(d)


def _scale_body(u_ref, dA_ref, dB_ref, g_ref):
    g = u_ref[...] * _dinv(dA_ref, dB_ref)
    g_ref[0] = g[:, :_HW]
    g_ref[1] = g[:, _HW:]


def _l1_body(aggA, aggB, gA, gB, dA, dB, b1_ref, w2_ref, g2_ref):
    dinv = _dinv(dA, dB)
    agg = jnp.concatenate([aggA[...], aggB[...]], axis=1)
    g = jnp.concatenate([gA[...], gB[...]], axis=1)
    h1 = jax.nn.relu((agg + g) * dinv + b1_ref[...])
    v = jnp.dot(h1, w2_ref[...], preferred_element_type=jnp.float32)
    g2_ref[...] = v * dinv


def _fin_body(a2A, a2B, g2, dA, dB, b2_ref, wl_ref, bl_ref, o_ref):
    dinv = _dinv(dA, dB)
    h2 = (a2A[...] + a2B[...] + g2[...]) * dinv + b2_ref[...]
    flat = jnp.concatenate([h2 * h2[:, j:j + 1] for j in range(C)], axis=1)
    maxnorm = 1.0 - EPS
    norm = jnp.maximum(jnp.sqrt(jnp.sum(flat * flat, axis=1, keepdims=True)),
                       MIN_NORM)
    hy = jnp.where(norm > maxnorm, flat / norm * maxnorm, flat)
    pn = jnp.maximum(jnp.sqrt(jnp.sum(hy * hy, axis=1, keepdims=True)),
                     MIN_NORM)
    pc = jnp.clip(pn, -1.0 + 1e-7, 1.0 - 1e-7)
    atanh = 0.5 * jnp.log((1.0 + pc) / (1.0 - pc))
    he = (atanh / pn) * hy
    z = jnp.dot(he, wl_ref[...], preferred_element_type=jnp.float32) \
        + bl_ref[...]
    un = jnp.maximum(jnp.sqrt(jnp.sum(z * z, axis=1, keepdims=True)),
                     MIN_NORM)
    e = jnp.tanh(un) * z / un
    n2 = jnp.maximum(jnp.sqrt(jnp.sum(e * e, axis=1, keepdims=True)),
                     MIN_NORM)
    hh = jnp.where(n2 > maxnorm, e / n2 * maxnorm, e)
    m = jnp.max(hh, axis=1, keepdims=True)
    sh = hh - m
    lse = jnp.log(jnp.sum(jnp.exp(sh), axis=1, keepdims=True))
    o_ref[...] = sh - lse


def _row_spec(w):
    return pl.BlockSpec((_BR, w), lambda i: (i, 0))


def _rowB_spec(w):
    return pl.BlockSpec((_BR, w), lambda i: (i + _GRID, 0))


def _full_spec(a, b):
    return pl.BlockSpec((a, b), lambda i: (0, 0))


def kernel(x, edge_index, W1, b1, W2, b2, Wl, bl):
    src = edge_index[0].astype(jnp.int32)
    dst = edge_index[1].astype(jnp.int32)
    pad = EP - E
    src_p = jnp.concatenate([src, jnp.zeros((pad,), jnp.int32)])
    dst_p = jnp.concatenate([dst, jnp.full((pad,), N, jnp.int32)])
    src2_r = jnp.concatenate([src_p, src_p + N]).reshape(NC, NS, NB1, EB1)
    dst1_r = dst_p.reshape(NS, NB1, EB1)
    src2w_r = src_p.reshape(NC * NS, NB2, EB)
    dst2w_r = dst_p.reshape(NC * NS, NB2, EB)

    ones_d = jnp.ones((EB, C), jnp.float32)
    zeros_c = jnp.zeros((SLAB, C), jnp.float32)

    deg2 = _sc_deg(dst2w_r, ones_d, zeros_c)

    u = pl.pallas_call(
        _mm_body, grid=(_GRID,),
        in_specs=[_row_spec(DF), _full_spec(DF, H)],
        out_specs=_row_spec(H),
        out_shape=jax.ShapeDtypeStruct((N, H), jnp.float32),
    )(x, W1)

    g_st = pl.pallas_call(
        _scale_body, grid=(_GRID,),
        in_specs=[_row_spec(H), _row_spec(C), _rowB_spec(C)],
        out_specs=pl.BlockSpec((NC, _BR, _HW), lambda i: (0, i, 0)),
        out_shape=jax.ShapeDtypeStruct((NC, N, _HW), jnp.float32),
    )(u, deg2, deg2)
    g_st = g_st.reshape(NC * N, _HW)

    agg1 = _sc_agg1(g_st, src2_r, dst1_r)

    g2 = pl.pallas_call(
        _l1_body, grid=(_GRID,),
        in_specs=[_row_spec(_HW), _rowB_spec(_HW), _row_spec(_HW),
                  _rowB_spec(_HW), _row_spec(C), _rowB_spec(C),
                  _full_spec(1, H), _full_spec(H, C)],
        out_specs=_row_spec(C),
        out_shape=jax.ShapeDtypeStruct((N, C), jnp.float32),
    )(agg1, agg1, g_st, g_st, deg2, deg2, b1.reshape(1, H), W2)

    agg2 = _sc_agg2(g2, src2w_r, dst2w_r, zeros_c)

    out = pl.pallas_call(
        _fin_body, grid=(_GRID,),
        in_specs=[_row_spec(C), _rowB_spec(C), _row_spec(C), _row_spec(C),
                  _rowB_spec(C), _full_spec(1, C), _full_spec(C * C, C),
                  _full_spec(1, C)],
        out_specs=_row_spec(C),
        out_shape=jax.ShapeDtypeStruct((N, C), jnp.float32),
    )(agg2, agg2, g2, deg2, deg2, b2.reshape(1, C), Wl, bl.reshape(1, C))
    return out


# speedup vs baseline: 1.2209x; 1.0593x over previous
import functools

import jax
import jax.numpy as jnp
from jax import lax
from jax.experimental import pallas as pl
from jax.experimental.pallas import tpu as pltpu
from jax.experimental.pallas import tpu_sc as plsc

N = 10000
E = 160000
DF = 256
H = 256
C = 16
MIN_NORM = 1e-15
EPS = 4e-3

NC = 2
NS = 16
SLAB = 624
TAIL_OFF = NS * SLAB
TAIL = N - TAIL_OFF

EB = 128
EB1 = 64
EP = 163840
NP = 10016
NB1 = EP // NS // EB1
NBH = NB1 // 4
NB2 = EP // (NC * NS) // EB
_HW = 128

_MESH = plsc.VectorSubcoreMesh(core_axis_name="c", subcore_axis_name="s")


def _zero_acc(zeros_hbm, acc_sh, s):
    pltpu.sync_copy(zeros_hbm.at[pl.ds(0, SLAB)],
                    acc_sh.at[pl.ds(s * SLAB, SLAB)])

    @pl.when(s == NS - 1)
    def _():
        pltpu.sync_copy(zeros_hbm.at[pl.ds(0, NP - TAIL_OFF)],
                        acc_sh.at[pl.ds(TAIL_OFF, NP - TAIL_OFF)])


def _write_out(acc_sh, out_hbm, c, s):
    pltpu.sync_copy(acc_sh.at[pl.ds(s * SLAB, SLAB)],
                    out_hbm.at[pl.ds(c * N + s * SLAB, SLAB)])

    @pl.when(s == NS - 1)
    def _():
        pltpu.sync_copy(acc_sh.at[pl.ds(TAIL_OFF, TAIL)],
                        out_hbm.at[pl.ds(c * N + TAIL_OFF, TAIL)])


@functools.partial(
    pl.kernel,
    out_type=jax.ShapeDtypeStruct((NC * N, C), jnp.float32),
    mesh=_MESH,
    scratch_types=[
        pltpu.VMEM((NB2, EB), jnp.int32),
        pltpu.VMEM((EB, C), jnp.float32),
        pltpu.VMEM_SHARED((NP, C), jnp.float32),
    ],
    compiler_params=pltpu.CompilerParams(use_tc_tiling_on_sc=False),
)
def _sc_deg(dst_hbm, ones_hbm, zeros_hbm, out_hbm, dst_v, ones_v, acc_sh):
    c = lax.axis_index("c")
    s = lax.axis_index("s")
    _zero_acc(zeros_hbm, acc_sh, s)
    pltpu.sync_copy(ones_hbm, ones_v)
    pltpu.sync_copy(dst_hbm.at[c * NS + s], dst_v)
    plsc.subcore_barrier()

    def body(j, carry):
        pltpu.sync_copy(ones_v, acc_sh.at[dst_v.at[j]], add=True)
        return carry

    lax.fori_loop(0, NB2, body, 0)
    plsc.subcore_barrier()
    _write_out(acc_sh, out_hbm, c, s)


@functools.partial(
    pl.kernel,
    out_type=jax.ShapeDtypeStruct((NC * N, _HW), jnp.float32),
    mesh=_MESH,
    scratch_types=[
        pltpu.VMEM((NBH, EB1), jnp.int32),
        pltpu.VMEM((NBH, EB1), jnp.int32),
        pltpu.VMEM((EB1, _HW), jnp.float32),
        pltpu.VMEM((EB1, _HW), jnp.float32),
        pltpu.VMEM((EB1, _HW), jnp.float32),
        pltpu.VMEM((EB1, _HW), jnp.float32),
        pltpu.VMEM_SHARED((NP, _HW), jnp.float32),
        pltpu.SemaphoreType.DMA,
        pltpu.SemaphoreType.DMA,
        pltpu.SemaphoreType.DMA,
        pltpu.SemaphoreType.DMA,
    ],
)
def _sc_agg1(g_hbm, src2_hbm, dst_hbm, out_hbm,
             src_v, dst_v, b0, b1, b2, b3, acc_sh, s0, s1, s2, s3):
    c = lax.axis_index("c")
    s = lax.axis_index("s")
    zv = jnp.zeros((16,), jnp.float32)

    def zrow(i, carry):
        for k in range(8):
            b0[i, pl.ds(k * 16, 16)] = zv
        return carry

    lax.fori_loop(0, EB1, zrow, 0)
    for off in range(0, 576, EB1):
        pltpu.sync_copy(b0, acc_sh.at[pl.ds(s * SLAB + off, EB1)])
    pltpu.sync_copy(b0.at[pl.ds(0, SLAB - 576)],
                    acc_sh.at[pl.ds(s * SLAB + 576, SLAB - 576)])

    @pl.when(s == NS - 1)
    def _():
        pltpu.sync_copy(b0.at[pl.ds(0, NP - TAIL_OFF)],
                        acc_sh.at[pl.ds(TAIL_OFF, NP - TAIL_OFF)])

    plsc.subcore_barrier()

    bufs = (b0, b1, b2, b3)
    sems = (s0, s1, s2, s3)

    def body(i, carry):
        for k in range(4):
            j = 4 * i + k
            buf, sem = bufs[k], sems[k]
            pltpu.make_async_copy(g_hbm.at[src_v.at[j]], buf, sem).wait()
            pltpu.sync_copy(buf, acc_sh.at[dst_v.at[j]], add=True)
            nj = j + 4

            @pl.when(nj < NBH)
            def _():
                pltpu.async_copy(g_hbm.at[src_v.at[nj]], buf, sem)

        return carry

    for hf in range(4):
        pltpu.sync_copy(src2_hbm.at[c, s, pl.ds(hf * NBH, NBH)], src_v)
        pltpu.sync_copy(dst_hbm.at[s, pl.ds(hf * NBH, NBH)], dst_v)
        for k in range(4):
            pltpu.async_copy(g_hbm.at[src_v.at[k]], bufs[k], sems[k])
        lax.fori_loop(0, NBH // 4, body, 0)
    plsc.subcore_barrier()
    _write_out(acc_sh, out_hbm, c, s)


@functools.partial(
    pl.kernel,
    out_type=jax.ShapeDtypeStruct((NC * N, C), jnp.float32),
    mesh=_MESH,
    scratch_types=[
        pltpu.VMEM((NB2, EB), jnp.int32),
        pltpu.VMEM((NB2, EB), jnp.int32),
        pltpu.VMEM((EB, C), jnp.float32),
        pltpu.VMEM((EB, C), jnp.float32),
        pltpu.VMEM_SHARED((NP, C), jnp.float32),
        pltpu.SemaphoreType.DMA,
        pltpu.SemaphoreType.DMA,
    ],
    compiler_params=pltpu.CompilerParams(use_tc_tiling_on_sc=False),
)
def _sc_agg2(g2_hbm, src_hbm, dst_hbm, zeros_hbm, out_hbm,
             src_v, dst_v, buf_a, buf_b, acc_sh, sem_a, sem_b):
    c = lax.axis_index("c")
    s = lax.axis_index("s")
    _zero_acc(zeros_hbm, acc_sh, s)
    w = c * NS + s
    pltpu.sync_copy(src_hbm.at[w], src_v)
    pltpu.sync_copy(dst_hbm.at[w], dst_v)
    plsc.subcore_barrier()

    def body(i, carry):
        ja = 2 * i
        jb = 2 * i + 1
        da = pltpu.async_copy(g2_hbm.at[src_v.at[ja]], buf_a, sem_a)
        db = pltpu.async_copy(g2_hbm.at[src_v.at[jb]], buf_b, sem_b)
        da.wait()
        pltpu.sync_copy(buf_a, acc_sh.at[dst_v.at[ja]], add=True)
        db.wait()
        pltpu.sync_copy(buf_b, acc_sh.at[dst_v.at[jb]], add=True)
        return carry

    lax.fori_loop(0, NB2 // 2, body, 0)
    plsc.subcore_barrier()
    _write_out(acc_sh, out_hbm, c, s)


_BR = 1000
_GRID = N // _BR


def _mm_body(x_ref, w_ref, o_ref):
    o_ref[...] = jnp.dot(x_ref[...], w_ref[...],
                         preferred_element_type=jnp.float32)


def _dinv(dA_ref, dB_ref):
    d = 1.0 + dA_ref[:, :1] + dB_ref[:, :1]
    return lax.rsqrt(d)


def _scale_body(u_ref, dA_ref, dB_ref, g_ref):
    g = u_ref[...] * _dinv(dA_ref, dB_ref)
    g_ref[0] = g[:, :_HW]
    g_ref[1] = g[:, _HW:]


def _l1_body(aggA, aggB, gA, gB, dA, dB, b1_ref, w2_ref, g2_ref):
    dinv = _dinv(dA, dB)
    agg = jnp.concatenate([aggA[...], aggB[...]], axis=1)
    g = jnp.concatenate([gA[...], gB[...]], axis=1)
    h1 = jax.nn.relu((agg + g) * dinv + b1_ref[...])
    v = jnp.dot(h1, w2_ref[...], preferred_element_type=jnp.float32)
    g2_ref[...] = v * dinv


def _fin_body(a2A, a2B, g2, dA, dB, b2_ref, wl_ref, bl_ref, o_ref):
    dinv = _dinv(dA, dB)
    h2 = (a2A[...] + a2B[...] + g2[...]) * dinv + b2_ref[...]
    rowj = lax.broadcasted_iota(jnp.int32, (C, C * C), 0)
    colc = lax.broadcasted_iota(jnp.int32, (C, C * C), 1)
    t_sel = (rowj == colc % C).astype(jnp.float32)
    r_sel = (rowj == colc // C).astype(jnp.float32)
    flat = (jnp.dot(h2, t_sel, preferred_element_type=jnp.float32,
                    precision=lax.Precision.HIGHEST)
            * jnp.dot(h2, r_sel, preferred_element_type=jnp.float32,
                      precision=lax.Precision.HIGHEST))
    maxnorm = 1.0 - EPS
    norm = jnp.maximum(jnp.sqrt(jnp.sum(flat * flat, axis=1, keepdims=True)),
                       MIN_NORM)
    hy = jnp.where(norm > maxnorm, flat / norm * maxnorm, flat)
    pn = jnp.maximum(jnp.sqrt(jnp.sum(hy * hy, axis=1, keepdims=True)),
                     MIN_NORM)
    pc = jnp.clip(pn, -1.0 + 1e-7, 1.0 - 1e-7)
    atanh = 0.5 * jnp.log((1.0 + pc) / (1.0 - pc))
    he = (atanh / pn) * hy
    z = jnp.dot(he, wl_ref[...], preferred_element_type=jnp.float32) \
        + bl_ref[...]
    un = jnp.maximum(jnp.sqrt(jnp.sum(z * z, axis=1, keepdims=True)),
                     MIN_NORM)
    e = jnp.tanh(un) * z / un
    n2 = jnp.maximum(jnp.sqrt(jnp.sum(e * e, axis=1, keepdims=True)),
                     MIN_NORM)
    hh = jnp.where(n2 > maxnorm, e / n2 * maxnorm, e)
    m = jnp.max(hh, axis=1, keepdims=True)
    sh = hh - m
    lse = jnp.log(jnp.sum(jnp.exp(sh), axis=1, keepdims=True))
    o_ref[...] = sh - lse


def _row_spec(w):
    return pl.BlockSpec((_BR, w), lambda i: (i, 0))


def _rowB_spec(w):
    return pl.BlockSpec((_BR, w), lambda i: (i + _GRID, 0))


def _full_spec(a, b):
    return pl.BlockSpec((a, b), lambda i: (0, 0))


def kernel(x, edge_index, W1, b1, W2, b2, Wl, bl):
    src = edge_index[0].astype(jnp.int32)
    dst = edge_index[1].astype(jnp.int32)
    pad = EP - E
    src_p = jnp.concatenate([src, jnp.zeros((pad,), jnp.int32)])
    dst_p = jnp.concatenate([dst, jnp.full((pad,), N, jnp.int32)])
    src2_r = jnp.concatenate([src_p, src_p + N]).reshape(NC, NS, NB1, EB1)
    dst1_r = dst_p.reshape(NS, NB1, EB1)
    src2w_r = src_p.reshape(NC * NS, NB2, EB)
    dst2w_r = dst_p.reshape(NC * NS, NB2, EB)

    ones_d = jnp.ones((EB, C), jnp.float32)
    zeros_c = jnp.zeros((SLAB, C), jnp.float32)

    deg2 = _sc_deg(dst2w_r, ones_d, zeros_c)

    u = pl.pallas_call(
        _mm_body, grid=(_GRID,),
        in_specs=[_row_spec(DF), _full_spec(DF, H)],
        out_specs=_row_spec(H),
        out_shape=jax.ShapeDtypeStruct((N, H), jnp.float32),
    )(x, W1)

    g_st = pl.pallas_call(
        _scale_body, grid=(_GRID,),
        in_specs=[_row_spec(H), _row_spec(C), _rowB_spec(C)],
        out_specs=pl.BlockSpec((NC, _BR, _HW), lambda i: (0, i, 0)),
        out_shape=jax.ShapeDtypeStruct((NC, N, _HW), jnp.float32),
    )(u, deg2, deg2)
    g_st = g_st.reshape(NC * N, _HW)

    agg1 = _sc_agg1(g_st, src2_r, dst1_r)

    g2 = pl.pallas_call(
        _l1_body, grid=(_GRID,),
        in_specs=[_row_spec(_HW), _rowB_spec(_HW), _row_spec(_HW),
                  _rowB_spec(_HW), _row_spec(C), _rowB_spec(C),
                  _full_spec(1, H), _full_spec(H, C)],
        out_specs=_row_spec(C),
        out_shape=jax.ShapeDtypeStruct((N, C), jnp.float32),
    )(agg1, agg1, g_st, g_st, deg2, deg2, b1.reshape(1, H), W2)

    agg2 = _sc_agg2(g2, src2w_r, dst2w_r, zeros_c)

    out = pl.pallas_call(
        _fin_body, grid=(_GRID,),
        in_specs=[_row_spec(C), _rowB_spec(C), _row_spec(C), _row_spec(C),
                  _rowB_spec(C), _full_spec(1, C), _full_spec(C * C, C),
                  _full_spec(1, C)],
        out_specs=_row_spec(C),
        out_shape=jax.ShapeDtypeStruct((N, C), jnp.float32),
    )(agg2, agg2, g2, deg2, deg2, b2.reshape(1, C), Wl, bl.reshape(1, C))
    return out
